# baseline (speedup 1.0000x reference)
import functools
import math

import jax
import jax.numpy as jnp
from jax import lax
from jax.experimental import pallas as pl
from jax.experimental.pallas import tpu as pltpu
from jax.experimental.pallas import tpu_sc as plsc

D = 1024
H = 16
E = 64
K = 2
HID = 1024
DFF = 4096
NC_CLS = 1000
CAP = 192
S = 2048
DH = D // H
ROWB = 256
NRB = S // ROWB
ECAP = E * CAP

_BF = jnp.bfloat16
_DN = (((1,), (1,)), ((), ()))


def _ln(x, g, b):
    m = jnp.mean(x, axis=-1, keepdims=True)
    v = jnp.mean((x - m) ** 2, axis=-1, keepdims=True)
    return (x - m) * lax.rsqrt(v + 1e-5) * g + b


def _qkv_body(has_pe, x_ref, pe_ref, iw_ref, ib_ref, h_ref, q_ref, k_ref,
              v_ref, x_val=None):
    x = x_val if x_val is not None else x_ref[...]
    if has_pe:
        x = x + pe_ref[...]
    h_ref[...] = x
    xb = x.astype(_BF)
    iw = iw_ref[...].astype(_BF)
    q = lax.dot_general(xb, iw[0:D, :], _DN,
                        preferred_element_type=jnp.float32)
    q = (q + ib_ref[0:1, 0:D]) * (1.0 / math.sqrt(DH))
    q_ref[...] = q.astype(_BF)
    k_ref[...] = (lax.dot_general(xb, iw[D:2 * D, :], _DN,
                                  preferred_element_type=jnp.float32)
                  + ib_ref[0:1, D:2 * D]).astype(_BF)
    v_ref[...] = (lax.dot_general(xb, iw[2 * D:3 * D, :], _DN,
                                  preferred_element_type=jnp.float32)
                  + ib_ref[0:1, 2 * D:3 * D]).astype(_BF)


def _qkv(x, pe, iw, ib):
    has_pe = pe is not None
    args = (x,) + ((pe,) if has_pe else ()) + (iw, ib)
    pe_spec = ([pl.BlockSpec((ROWB, D), lambda i: (i, 0))] if has_pe else [])

    def body(*refs):
        if has_pe:
            _qkv_body(True, *refs)
        else:
            x_ref, iw_ref, ib_ref, h_ref, q_ref, k_ref, v_ref = refs
            _qkv_body(False, x_ref, None, iw_ref, ib_ref, h_ref, q_ref,
                      k_ref, v_ref)

    return pl.pallas_call(
        body,
        grid=(NRB,),
        in_specs=[pl.BlockSpec((ROWB, D), lambda i: (i, 0))] + pe_spec + [
            pl.BlockSpec((3 * D, D), lambda i: (0, 0)),
            pl.BlockSpec((1, 3 * D), lambda i: (0, 0)),
        ],
        out_specs=[pl.BlockSpec((ROWB, D), lambda i: (i, 0))] * 4,
        out_shape=[jax.ShapeDtypeStruct((S, D), jnp.float32)]
        + [jax.ShapeDtypeStruct((S, D), _BF)] * 3,
    )(*args)


RB_A = 256
NRB_A = S // RB_A


def _attn_proj_ln_body(q_ref, k_ref, v_ref, ow_ref, ob_ref, res_ref, g_ref,
                       b_ref, out_ref):
    parts = []
    for h in range(H):
        hs = slice(h * DH, (h + 1) * DH)
        q = q_ref[:, hs]
        k = k_ref[:, hs]
        v = v_ref[:, hs]
        s = lax.dot_general(q, k, _DN,
                            preferred_element_type=jnp.float32)
        m = jnp.max(s, axis=1, keepdims=True)
        e = jnp.exp(s - m).astype(_BF)
        ones = jnp.ones((S, 1), _BF)
        r = 1.0 / lax.dot_general(e, ones, (((1,), (0,)), ((), ())),
                                  preferred_element_type=jnp.float32)
        ov = lax.dot_general(e, v, (((1,), (0,)), ((), ())),
                             preferred_element_type=jnp.float32)
        parts.append(ov * r)
    o = jnp.concatenate(parts, axis=1).astype(_BF)
    p = lax.dot_general(o, ow_ref[...].astype(_BF), _DN,
                        preferred_element_type=jnp.float32) + ob_ref[...]
    out_ref[...] = _ln(res_ref[...] + p, g_ref[...], b_ref[...])


def _attn_proj_ln(q, k, v, ow, ob, res, g, b):
    return pl.pallas_call(
        _attn_proj_ln_body,
        grid=(NRB_A,),
        in_specs=[
            pl.BlockSpec((RB_A, D), lambda i: (i, 0)),
            pl.BlockSpec((S, D), lambda i: (0, 0)),
            pl.BlockSpec((S, D), lambda i: (0, 0)),
            pl.BlockSpec((D, D), lambda i: (0, 0)),
            pl.BlockSpec((1, D), lambda i: (0, 0)),
            pl.BlockSpec((RB_A, D), lambda i: (i, 0)),
            pl.BlockSpec((1, D), lambda i: (0, 0)),
            pl.BlockSpec((1, D), lambda i: (0, 0)),
        ],
        out_specs=pl.BlockSpec((RB_A, D), lambda i: (i, 0)),
        out_shape=jax.ShapeDtypeStruct((S, D), jnp.float32),
    )(q, k, v, ow, ob, res, g, b)


def _router_body(h_ref, rw_ref, rb_ref, ds_ref, dg_ref, w_ref):
    def blk(i, counts):
        hb = h_ref[pl.ds(i * ROWB, ROWB), :]
        logits = lax.dot_general(hb, rw_ref[...], _DN,
                                 preferred_element_type=jnp.float32) \
            + rb_ref[...]
        iota = lax.broadcasted_iota(jnp.int32, (ROWB, E), 1)
        m1 = jnp.max(logits, axis=1, keepdims=True)
        e0 = jnp.min(jnp.where(logits == m1, iota, E), axis=1, keepdims=True)
        l2 = jnp.where(iota == e0, -1e30, logits)
        m2 = jnp.max(l2, axis=1, keepdims=True)
        e1 = jnp.min(jnp.where(l2 == m2, iota, E), axis=1, keepdims=True)
        z = jnp.exp(m2 - m1)
        p0 = 1.0 / (1.0 + z)
        p1 = z / (1.0 + z)

        oh0 = (iota == e0).astype(jnp.float32)
        oh1 = (iota == e1).astype(jnp.float32)
        hh = oh0 + oh1
        r = lax.broadcasted_iota(jnp.int32, (ROWB, ROWB), 0)
        c = lax.broadcasted_iota(jnp.int32, (ROWB, ROWB), 1)
        tri = (r > c).astype(jnp.float32)
        cum = lax.dot_general(tri, hh, (((1,), (0,)), ((), ())),
                              preferred_element_type=jnp.float32) + counts
        pos0 = jnp.sum(cum * oh0, axis=1, keepdims=True).astype(jnp.int32)
        pos1 = jnp.sum(cum * oh1, axis=1, keepdims=True).astype(jnp.int32)

        d0 = e0 * CAP + pos0
        d1 = e1 * CAP + pos1
        v0 = pos0 < CAP
        v1 = pos1 < CAP
        sl = pl.ds(i * ROWB, ROWB)
        ds_ref[sl, 0:1] = jnp.where(v0, d0, ECAP)
        ds_ref[sl, 1:2] = jnp.where(v1, d1, ECAP)
        dg_ref[sl, 0:1] = jnp.where(v0, d0, 0)
        dg_ref[sl, 1:2] = jnp.where(v1, d1, 0)
        w_ref[sl, 0:1] = jnp.where(v0, p0, 0.0)
        w_ref[sl, 1:2] = jnp.where(v1, p1, 0.0)
        return counts + jnp.sum(hh, axis=0, keepdims=True)

    lax.fori_loop(0, NRB, blk, jnp.zeros((1, E), jnp.float32))


def _router(h, rw, rb):
    return pl.pallas_call(
        _router_body,
        grid=(1,),
        in_specs=[
            pl.BlockSpec((S, D), lambda i: (0, 0)),
            pl.BlockSpec((E, D), lambda i: (0, 0)),
            pl.BlockSpec((1, E), lambda i: (0, 0)),
        ],
        out_specs=[pl.BlockSpec((S, K), lambda i: (0, 0))] * 3,
        out_shape=[
            jax.ShapeDtypeStruct((S, K), jnp.int32),
            jax.ShapeDtypeStruct((S, K), jnp.int32),
            jax.ShapeDtypeStruct((S, K), jnp.float32),
        ],
    )(h, rw, rb)


def _moe_ffn_body(x_ref, w1_ref, b1_ref, w2_ref, b2_ref, y_ref):
    x = x_ref[...].astype(_BF)
    hdn = lax.dot_general(x, w1_ref[...].astype(_BF), _DN,
                          preferred_element_type=jnp.float32) + b1_ref[...]
    hdn = jnp.maximum(hdn, 0.0).astype(_BF)
    y_ref[...] = lax.dot_general(hdn, w2_ref[...].astype(_BF), _DN,
                                 preferred_element_type=jnp.float32) \
        + b2_ref[...]


def _moe_ffn(buf, w1, b1, w2, b2):
    return pl.pallas_call(
        _moe_ffn_body,
        grid=(E,),
        in_specs=[
            pl.BlockSpec((CAP, D), lambda e: (e, 0)),
            pl.BlockSpec((None, HID, D), lambda e: (e, 0, 0)),
            pl.BlockSpec((None, 1, HID), lambda e: (e, 0, 0)),
            pl.BlockSpec((None, D, HID), lambda e: (e, 0, 0)),
            pl.BlockSpec((None, 1, D), lambda e: (e, 0, 0)),
        ],
        out_specs=pl.BlockSpec((CAP, D), lambda e: (e, 0)),
        out_shape=jax.ShapeDtypeStruct((ECAP, D), jnp.float32),
    )(buf, w1, b1, w2, b2)


def _combine_qkv_body(h_ref, y0_ref, y1_ref, w_ref, g_ref, b_ref, iw_ref,
                      ib_ref, h2_ref, q_ref, k_ref, v_ref):
    w = w_ref[...]
    w0 = w[:, 0:1]
    w1 = w[:, 1:2]
    mo = jnp.where(w0 > 0, w0 * y0_ref[...], 0.0) \
        + jnp.where(w1 > 0, w1 * y1_ref[...], 0.0)
    h2 = _ln(h_ref[...] + mo, g_ref[...], b_ref[...])
    _qkv_body(False, None, None, iw_ref, ib_ref, h2_ref, q_ref, k_ref,
              v_ref, x_val=h2)


def _combine_qkv(h, ypair, w, g, b, iw, ib):
    return pl.pallas_call(
        _combine_qkv_body,
        grid=(NRB,),
        in_specs=[
            pl.BlockSpec((ROWB, D), lambda i: (i, 0)),
            pl.BlockSpec((ROWB, D), lambda i: (i, 0)),
            pl.BlockSpec((ROWB, D), lambda i: (i + NRB, 0)),
            pl.BlockSpec((ROWB, K), lambda i: (i, 0)),
            pl.BlockSpec((1, D), lambda i: (0, 0)),
            pl.BlockSpec((1, D), lambda i: (0, 0)),
            pl.BlockSpec((3 * D, D), lambda i: (0, 0)),
            pl.BlockSpec((1, 3 * D), lambda i: (0, 0)),
        ],
        out_specs=[pl.BlockSpec((ROWB, D), lambda i: (i, 0))] * 4,
        out_shape=[jax.ShapeDtypeStruct((S, D), jnp.float32)]
        + [jax.ShapeDtypeStruct((S, D), _BF)] * 3,
    )(h, ypair, ypair, w, g, b, iw, ib)


def _ffn_pool_cls_body(h_ref, w1_ref, b1_ref, w2_ref, b2_ref, g_ref, b_ref,
                       cw_ref, cb_ref, out_ref, acc_ref):
    i = pl.program_id(0)
    x = h_ref[...]
    hdn = lax.dot_general(x.astype(_BF), w1_ref[...].astype(_BF), _DN,
                          preferred_element_type=jnp.float32) + b1_ref[...]
    hdn = jnp.maximum(hdn, 0.0).astype(_BF)
    f = lax.dot_general(hdn, w2_ref[...].astype(_BF), _DN,
                        preferred_element_type=jnp.float32) + b2_ref[...]
    h4 = _ln(x + f, g_ref[...], b_ref[...])
    part = jnp.sum(h4, axis=0, keepdims=True)

    @pl.when(i == 0)
    def _():
        acc_ref[...] = part

    @pl.when(i > 0)
    def _():
        acc_ref[...] = acc_ref[...] + part

    @pl.when(i == NRB - 1)
    def _():
        pooled = acc_ref[...] * (1.0 / S)
        out_ref[...] = lax.dot_general(
            pooled, cw_ref[...], _DN,
            preferred_element_type=jnp.float32) + cb_ref[...]


def _ffn_pool_cls(h, w1, b1, w2, b2, g, b, cw, cb):
    return pl.pallas_call(
        _ffn_pool_cls_body,
        grid=(NRB,),
        in_specs=[
            pl.BlockSpec((ROWB, D), lambda i: (i, 0)),
            pl.BlockSpec((DFF, D), lambda i: (0, 0)),
            pl.BlockSpec((1, DFF), lambda i: (0, 0)),
            pl.BlockSpec((D, DFF), lambda i: (0, 0)),
            pl.BlockSpec((1, D), lambda i: (0, 0)),
            pl.BlockSpec((1, D), lambda i: (0, 0)),
            pl.BlockSpec((1, D), lambda i: (0, 0)),
            pl.BlockSpec((NC_CLS, D), lambda i: (0, 0)),
            pl.BlockSpec((1, NC_CLS), lambda i: (0, 0)),
        ],
        out_specs=pl.BlockSpec((1, NC_CLS), lambda i: (0, 0)),
        out_shape=jax.ShapeDtypeStruct((1, NC_CLS), jnp.float32),
        scratch_shapes=[pltpu.VMEM((1, D), jnp.float32)],
    )(h, w1, b1, w2, b2, g, b, cw, cb)


_SC_INFO = plsc.get_sparse_core_info()
_SC_NC = _SC_INFO.num_cores
_SC_NS = _SC_INFO.num_subcores
_NW = _SC_NC * _SC_NS

_TBL = ECAP


_DCH = 48
_DPW = ECAP // _NW
_DNCH = _DPW // _DCH


def _dispatch_body(d_hbm, h_hbm, buf_hbm, tbl_hbm, d_v, tbl_v, idx_a,
                   idx_b, rows_a, rows_b, sem_a, sem_b):
    cid = lax.axis_index("c")
    sid = lax.axis_index("s")
    wid = sid * _SC_NC + cid

    @pl.when(sid == 0)
    def _():
        pltpu.sync_copy(d_hbm, d_v)

        def init(i, _):
            tbl_v[pl.ds(i * 16, 16)] = \
                (lax.iota(jnp.int32, 16) + i * 16) & (S - 1)
            return 0

        lax.fori_loop(0, _TBL // 16, init, 0)

        def scat(i, _):
            dv = d_v[pl.ds(i * 16, 16)]
            tv = (lax.iota(jnp.int32, 16) + i * 16) & (S - 1)
            plsc.store_scatter(tbl_v, [dv], tv, mask=dv < _TBL)
            return 0

        lax.fori_loop(0, (S * K) // 16, scat, 0)
        pltpu.sync_copy(tbl_v, tbl_hbm)

    plsc.subcore_barrier()

    base = wid * _DPW
    bufs = [(idx_a, rows_a, sem_a), (idx_b, rows_b, sem_b)]
    pltpu.sync_copy(tbl_hbm.at[pl.ds(base, _DCH)], idx_a)
    cur = pltpu.async_copy(h_hbm.at[idx_a], rows_a, sem_a)
    for c in range(_DNCH):
        _, rows_c, _ = bufs[c % 2]
        nxt = None
        if c + 1 < _DNCH:
            idx_n, rows_n, sem_n = bufs[(c + 1) % 2]
            pltpu.sync_copy(tbl_hbm.at[pl.ds(base + (c + 1) * _DCH, _DCH)],
                            idx_n)
            nxt = pltpu.async_copy(h_hbm.at[idx_n], rows_n, sem_n)
        cur.wait()
        pltpu.sync_copy(rows_c, buf_hbm.at[pl.ds(base + c * _DCH, _DCH)])
        cur = nxt


_dispatch_call = pl.kernel(
    _dispatch_body,
    out_type=(jax.ShapeDtypeStruct((ECAP, D), jnp.float32),
              jax.ShapeDtypeStruct((_TBL,), jnp.int32)),
    mesh=plsc.VectorSubcoreMesh(core_axis_name="c", subcore_axis_name="s"),
    compiler_params=pltpu.CompilerParams(needs_layout_passes=False),
    scratch_types=[
        pltpu.VMEM((S * K,), jnp.int32),
        pltpu.VMEM((_TBL,), jnp.int32),
        pltpu.VMEM((_DCH,), jnp.int32),
        pltpu.VMEM((_DCH,), jnp.int32),
        pltpu.VMEM((_DCH, D), jnp.float32),
        pltpu.VMEM((_DCH, D), jnp.float32),
        pltpu.SemaphoreType.DMA,
        pltpu.SemaphoreType.DMA,
    ],
)


def _make_gather(nrows, chunk):
    assert nrows % (_NW * chunk) == 0
    per_w = nrows // _NW
    nchunk = per_w // chunk

    def body(tbl_hbm, idx_hbm, out_hbm, idx_v, rows_v, sem):
        wid = lax.axis_index("s") * _SC_NC + lax.axis_index("c")
        base = wid * per_w

        def go(ci, _):
            off = base + ci * chunk
            pltpu.sync_copy(idx_hbm.at[pl.ds(off, chunk)], idx_v)
            pltpu.async_copy(tbl_hbm.at[idx_v], rows_v, sem).wait()
            pltpu.sync_copy(rows_v, out_hbm.at[pl.ds(off, chunk)])
            return 0

        lax.fori_loop(0, nchunk, go, 0)

    return pl.kernel(
        body,
        out_type=jax.ShapeDtypeStruct((nrows, D), jnp.float32),
        mesh=plsc.VectorSubcoreMesh(core_axis_name="c",
                                    subcore_axis_name="s"),
        scratch_types=[
            pltpu.VMEM((chunk,), jnp.int32),
            pltpu.VMEM((chunk, D), jnp.float32),
            pltpu.SemaphoreType.DMA,
        ],
    )


_gather_out = _make_gather(2 * S, 64)


def _pe_const(s, dm):
    pos = jnp.arange(s, dtype=jnp.float32)[:, None]
    div = jnp.exp(jnp.arange(0, dm, 2, dtype=jnp.float32)
                  * (-math.log(10000.0) / dm))
    pe = jnp.zeros((s, dm), dtype=jnp.float32)
    pe = pe.at[:, 0::2].set(jnp.sin(pos * div))
    pe = pe.at[:, 1::2].set(jnp.cos(pos * div))
    return pe


def kernel(x, a0_iw, a0_ib, a0_ow, a0_ob, n0a_g, n0a_b, r_w, r_b, e_w1,
           e_b1, e_w2, e_b2, n0b_g, n0b_b, a1_iw, a1_ib, a1_ow, a1_ob,
           n1a_g, n1a_b, l1_w, l1_b, l2_w, l2_b, n1b_g, n1b_b, c_w, c_b):
    xs = x.reshape(S, D)
    pe = _pe_const(S, D)
    row = lambda a: a.reshape(1, -1)

    h, q, k, v = _qkv(xs, pe, a0_iw, row(a0_ib))
    h1 = _attn_proj_ln(q, k, v, a0_ow, row(a0_ob), h, row(n0a_g),
                       row(n0a_b))

    ds, dg, w = _router(h1, r_w, row(r_b))
    buf, _ = _dispatch_call(ds.T.reshape(-1), h1)
    y = _moe_ffn(buf, e_w1, e_b1.reshape(E, 1, HID), e_w2,
                 e_b2.reshape(E, 1, D))
    ypair = _gather_out(y, dg.T.reshape(-1))

    h2, q, k, v = _combine_qkv(h1, ypair, w, row(n0b_g), row(n0b_b),
                               a1_iw, row(a1_ib))

    h3 = _attn_proj_ln(q, k, v, a1_ow, row(a1_ob), h2, row(n1a_g),
                       row(n1a_b))
    return _ffn_pool_cls(h3, l1_w, row(l1_b), l2_w, row(l2_b), row(n1b_g),
                         row(n1b_b), c_w, row(c_b))


# speedup vs baseline: 1.0546x; 1.0546x over previous
import functools
import math

import jax
import jax.numpy as jnp
from jax import lax
from jax.experimental import pallas as pl
from jax.experimental.pallas import tpu as pltpu
from jax.experimental.pallas import tpu_sc as plsc

D = 1024
H = 16
E = 64
K = 2
HID = 1024
DFF = 4096
NC_CLS = 1000
CAP = 192
S = 2048
DH = D // H
ROWB = 256
NRB = S // ROWB
ECAP = E * CAP

_BF = jnp.bfloat16
_DN = (((1,), (1,)), ((), ()))


def _ln(x, g, b):
    m = jnp.mean(x, axis=-1, keepdims=True)
    v = jnp.mean((x - m) ** 2, axis=-1, keepdims=True)
    return (x - m) * lax.rsqrt(v + 1e-5) * g + b


def _qkv_body(has_pe, x_ref, pe_ref, iw_ref, ib_ref, h_ref, q_ref, k_ref,
              v_ref, x_val=None):
    x = x_val if x_val is not None else x_ref[...]
    if has_pe:
        x = x + pe_ref[...]
    h_ref[...] = x
    xb = x.astype(_BF)
    iw = iw_ref[...].astype(_BF)
    q = lax.dot_general(xb, iw[0:D, :], _DN,
                        preferred_element_type=jnp.float32)
    q = (q + ib_ref[0:1, 0:D]) * (1.0 / math.sqrt(DH))
    q_ref[...] = q.astype(_BF)
    k_ref[...] = (lax.dot_general(xb, iw[D:2 * D, :], _DN,
                                  preferred_element_type=jnp.float32)
                  + ib_ref[0:1, D:2 * D]).astype(_BF)
    v_ref[...] = (lax.dot_general(xb, iw[2 * D:3 * D, :], _DN,
                                  preferred_element_type=jnp.float32)
                  + ib_ref[0:1, 2 * D:3 * D]).astype(_BF)


def _qkv(x, pe, iw, ib):
    has_pe = pe is not None
    args = (x,) + ((pe,) if has_pe else ()) + (iw, ib)
    pe_spec = ([pl.BlockSpec((ROWB, D), lambda i: (i, 0))] if has_pe else [])

    def body(*refs):
        if has_pe:
            _qkv_body(True, *refs)
        else:
            x_ref, iw_ref, ib_ref, h_ref, q_ref, k_ref, v_ref = refs
            _qkv_body(False, x_ref, None, iw_ref, ib_ref, h_ref, q_ref,
                      k_ref, v_ref)

    return pl.pallas_call(
        body,
        grid=(NRB,),
        in_specs=[pl.BlockSpec((ROWB, D), lambda i: (i, 0))] + pe_spec + [
            pl.BlockSpec((3 * D, D), lambda i: (0, 0)),
            pl.BlockSpec((1, 3 * D), lambda i: (0, 0)),
        ],
        out_specs=[pl.BlockSpec((ROWB, D), lambda i: (i, 0))] * 4,
        out_shape=[jax.ShapeDtypeStruct((S, D), jnp.float32)]
        + [jax.ShapeDtypeStruct((S, D), _BF)] * 3,
    )(*args)


RB_A = 256
NRB_A = S // RB_A


def _attn_proj_ln_body(q_ref, k_ref, v_ref, ow_ref, ob_ref, res_ref, g_ref,
                       b_ref, out_ref):
    parts = []
    for h in range(H):
        hs = slice(h * DH, (h + 1) * DH)
        q = q_ref[:, hs]
        k = k_ref[:, hs]
        v = v_ref[:, hs]
        s = lax.dot_general(q, k, _DN,
                            preferred_element_type=jnp.float32)
        m = jnp.max(s, axis=1, keepdims=True)
        e = jnp.exp(s - m)
        r = 1.0 / jnp.sum(e, axis=1, keepdims=True)
        ov = lax.dot_general(e.astype(_BF), v, (((1,), (0,)), ((), ())),
                             preferred_element_type=jnp.float32)
        parts.append(ov * r)
    o = jnp.concatenate(parts, axis=1).astype(_BF)
    p = lax.dot_general(o, ow_ref[...].astype(_BF), _DN,
                        preferred_element_type=jnp.float32) + ob_ref[...]
    out_ref[...] = _ln(res_ref[...] + p, g_ref[...], b_ref[...])


def _attn_proj_ln(q, k, v, ow, ob, res, g, b):
    return pl.pallas_call(
        _attn_proj_ln_body,
        grid=(NRB_A,),
        in_specs=[
            pl.BlockSpec((RB_A, D), lambda i: (i, 0)),
            pl.BlockSpec((S, D), lambda i: (0, 0)),
            pl.BlockSpec((S, D), lambda i: (0, 0)),
            pl.BlockSpec((D, D), lambda i: (0, 0)),
            pl.BlockSpec((1, D), lambda i: (0, 0)),
            pl.BlockSpec((RB_A, D), lambda i: (i, 0)),
            pl.BlockSpec((1, D), lambda i: (0, 0)),
            pl.BlockSpec((1, D), lambda i: (0, 0)),
        ],
        out_specs=pl.BlockSpec((RB_A, D), lambda i: (i, 0)),
        out_shape=jax.ShapeDtypeStruct((S, D), jnp.float32),
    )(q, k, v, ow, ob, res, g, b)


def _router_body(h_ref, rw_ref, rb_ref, ds_ref, dg_ref, w_ref):
    def blk(i, counts):
        hb = h_ref[pl.ds(i * ROWB, ROWB), :]
        logits = lax.dot_general(hb, rw_ref[...], _DN,
                                 preferred_element_type=jnp.float32) \
            + rb_ref[...]
        iota = lax.broadcasted_iota(jnp.int32, (ROWB, E), 1)
        m1 = jnp.max(logits, axis=1, keepdims=True)
        e0 = jnp.min(jnp.where(logits == m1, iota, E), axis=1, keepdims=True)
        l2 = jnp.where(iota == e0, -1e30, logits)
        m2 = jnp.max(l2, axis=1, keepdims=True)
        e1 = jnp.min(jnp.where(l2 == m2, iota, E), axis=1, keepdims=True)
        z = jnp.exp(m2 - m1)
        p0 = 1.0 / (1.0 + z)
        p1 = z / (1.0 + z)

        oh0 = (iota == e0).astype(jnp.float32)
        oh1 = (iota == e1).astype(jnp.float32)
        hh = oh0 + oh1
        r = lax.broadcasted_iota(jnp.int32, (ROWB, ROWB), 0)
        c = lax.broadcasted_iota(jnp.int32, (ROWB, ROWB), 1)
        tri = (r > c).astype(jnp.float32)
        cum = lax.dot_general(tri, hh, (((1,), (0,)), ((), ())),
                              preferred_element_type=jnp.float32) + counts
        pos0 = jnp.sum(cum * oh0, axis=1, keepdims=True).astype(jnp.int32)
        pos1 = jnp.sum(cum * oh1, axis=1, keepdims=True).astype(jnp.int32)

        d0 = e0 * CAP + pos0
        d1 = e1 * CAP + pos1
        v0 = pos0 < CAP
        v1 = pos1 < CAP
        sl = pl.ds(i * ROWB, ROWB)
        ds_ref[sl, 0:1] = jnp.where(v0, d0, ECAP)
        ds_ref[sl, 1:2] = jnp.where(v1, d1, ECAP)
        dg_ref[sl, 0:1] = jnp.where(v0, d0, 0)
        dg_ref[sl, 1:2] = jnp.where(v1, d1, 0)
        w_ref[sl, 0:1] = jnp.where(v0, p0, 0.0)
        w_ref[sl, 1:2] = jnp.where(v1, p1, 0.0)
        return counts + jnp.sum(hh, axis=0, keepdims=True)

    lax.fori_loop(0, NRB, blk, jnp.zeros((1, E), jnp.float32))


def _router(h, rw, rb):
    return pl.pallas_call(
        _router_body,
        grid=(1,),
        in_specs=[
            pl.BlockSpec((S, D), lambda i: (0, 0)),
            pl.BlockSpec((E, D), lambda i: (0, 0)),
            pl.BlockSpec((1, E), lambda i: (0, 0)),
        ],
        out_specs=[pl.BlockSpec((S, K), lambda i: (0, 0))] * 3,
        out_shape=[
            jax.ShapeDtypeStruct((S, K), jnp.int32),
            jax.ShapeDtypeStruct((S, K), jnp.int32),
            jax.ShapeDtypeStruct((S, K), jnp.float32),
        ],
    )(h, rw, rb)


def _moe_ffn_body(x_ref, w1_ref, b1_ref, w2_ref, b2_ref, y_ref):
    x = x_ref[...].astype(_BF)
    hdn = lax.dot_general(x, w1_ref[...].astype(_BF), _DN,
                          preferred_element_type=jnp.float32) + b1_ref[...]
    hdn = jnp.maximum(hdn, 0.0).astype(_BF)
    y_ref[...] = lax.dot_general(hdn, w2_ref[...].astype(_BF), _DN,
                                 preferred_element_type=jnp.float32) \
        + b2_ref[...]


def _moe_ffn(buf, w1, b1, w2, b2):
    return pl.pallas_call(
        _moe_ffn_body,
        grid=(E,),
        in_specs=[
            pl.BlockSpec((CAP, D), lambda e: (e, 0)),
            pl.BlockSpec((None, HID, D), lambda e: (e, 0, 0)),
            pl.BlockSpec((None, 1, HID), lambda e: (e, 0, 0)),
            pl.BlockSpec((None, D, HID), lambda e: (e, 0, 0)),
            pl.BlockSpec((None, 1, D), lambda e: (e, 0, 0)),
        ],
        out_specs=pl.BlockSpec((CAP, D), lambda e: (e, 0)),
        out_shape=jax.ShapeDtypeStruct((ECAP, D), jnp.float32),
    )(buf, w1, b1, w2, b2)


def _combine_qkv_body(h_ref, y0_ref, y1_ref, w_ref, g_ref, b_ref, iw_ref,
                      ib_ref, h2_ref, q_ref, k_ref, v_ref):
    w = w_ref[...]
    w0 = w[:, 0:1]
    w1 = w[:, 1:2]
    mo = jnp.where(w0 > 0, w0 * y0_ref[...], 0.0) \
        + jnp.where(w1 > 0, w1 * y1_ref[...], 0.0)
    h2 = _ln(h_ref[...] + mo, g_ref[...], b_ref[...])
    _qkv_body(False, None, None, iw_ref, ib_ref, h2_ref, q_ref, k_ref,
              v_ref, x_val=h2)


def _combine_qkv(h, ypair, w, g, b, iw, ib):
    return pl.pallas_call(
        _combine_qkv_body,
        grid=(NRB,),
        in_specs=[
            pl.BlockSpec((ROWB, D), lambda i: (i, 0)),
            pl.BlockSpec((ROWB, D), lambda i: (i, 0)),
            pl.BlockSpec((ROWB, D), lambda i: (i + NRB, 0)),
            pl.BlockSpec((ROWB, K), lambda i: (i, 0)),
            pl.BlockSpec((1, D), lambda i: (0, 0)),
            pl.BlockSpec((1, D), lambda i: (0, 0)),
            pl.BlockSpec((3 * D, D), lambda i: (0, 0)),
            pl.BlockSpec((1, 3 * D), lambda i: (0, 0)),
        ],
        out_specs=[pl.BlockSpec((ROWB, D), lambda i: (i, 0))] * 4,
        out_shape=[jax.ShapeDtypeStruct((S, D), jnp.float32)]
        + [jax.ShapeDtypeStruct((S, D), _BF)] * 3,
    )(h, ypair, ypair, w, g, b, iw, ib)


def _ffn_pool_cls_body(h_ref, w1_ref, b1_ref, w2_ref, b2_ref, g_ref, b_ref,
                       cw_ref, cb_ref, out_ref, acc_ref):
    i = pl.program_id(0)
    x = h_ref[...]
    hdn = lax.dot_general(x.astype(_BF), w1_ref[...].astype(_BF), _DN,
                          preferred_element_type=jnp.float32) + b1_ref[...]
    hdn = jnp.maximum(hdn, 0.0).astype(_BF)
    f = lax.dot_general(hdn, w2_ref[...].astype(_BF), _DN,
                        preferred_element_type=jnp.float32) + b2_ref[...]
    h4 = _ln(x + f, g_ref[...], b_ref[...])
    part = jnp.sum(h4, axis=0, keepdims=True)

    @pl.when(i == 0)
    def _():
        acc_ref[...] = part

    @pl.when(i > 0)
    def _():
        acc_ref[...] = acc_ref[...] + part

    @pl.when(i == NRB - 1)
    def _():
        pooled = acc_ref[...] * (1.0 / S)
        out_ref[...] = lax.dot_general(
            pooled, cw_ref[...], _DN,
            preferred_element_type=jnp.float32) + cb_ref[...]


def _ffn_pool_cls(h, w1, b1, w2, b2, g, b, cw, cb):
    return pl.pallas_call(
        _ffn_pool_cls_body,
        grid=(NRB,),
        in_specs=[
            pl.BlockSpec((ROWB, D), lambda i: (i, 0)),
            pl.BlockSpec((DFF, D), lambda i: (0, 0)),
            pl.BlockSpec((1, DFF), lambda i: (0, 0)),
            pl.BlockSpec((D, DFF), lambda i: (0, 0)),
            pl.BlockSpec((1, D), lambda i: (0, 0)),
            pl.BlockSpec((1, D), lambda i: (0, 0)),
            pl.BlockSpec((1, D), lambda i: (0, 0)),
            pl.BlockSpec((NC_CLS, D), lambda i: (0, 0)),
            pl.BlockSpec((1, NC_CLS), lambda i: (0, 0)),
        ],
        out_specs=pl.BlockSpec((1, NC_CLS), lambda i: (0, 0)),
        out_shape=jax.ShapeDtypeStruct((1, NC_CLS), jnp.float32),
        scratch_shapes=[pltpu.VMEM((1, D), jnp.float32)],
    )(h, w1, b1, w2, b2, g, b, cw, cb)


_SC_INFO = plsc.get_sparse_core_info()
_SC_NC = _SC_INFO.num_cores
_SC_NS = _SC_INFO.num_subcores
_NW = _SC_NC * _SC_NS

_TBL = ECAP


_DCH = 48
_DPW = ECAP // _NW
_DNCH = _DPW // _DCH


def _dispatch_body(d_hbm, h_hbm, buf_hbm, tbl_hbm, d_v, tbl_v, idx_a,
                   idx_b, rows_a, rows_b, sem_a, sem_b):
    cid = lax.axis_index("c")
    sid = lax.axis_index("s")
    wid = sid * _SC_NC + cid

    @pl.when(sid == 0)
    def _():
        pltpu.sync_copy(d_hbm, d_v)

        def init(i, _):
            tbl_v[pl.ds(i * 16, 16)] = \
                (lax.iota(jnp.int32, 16) + i * 16) & (S - 1)
            return 0

        lax.fori_loop(0, _TBL // 16, init, 0)

        def scat(i, _):
            dv = d_v[pl.ds(i * 16, 16)]
            tv = (lax.iota(jnp.int32, 16) + i * 16) & (S - 1)
            plsc.store_scatter(tbl_v, [dv], tv, mask=dv < _TBL)
            return 0

        lax.fori_loop(0, (S * K) // 16, scat, 0)
        pltpu.sync_copy(tbl_v, tbl_hbm)

    plsc.subcore_barrier()

    base = wid * _DPW
    bufs = [(idx_a, rows_a, sem_a), (idx_b, rows_b, sem_b)]
    pltpu.sync_copy(tbl_hbm.at[pl.ds(base, _DCH)], idx_a)
    cur = pltpu.async_copy(h_hbm.at[idx_a], rows_a, sem_a)
    for c in range(_DNCH):
        _, rows_c, _ = bufs[c % 2]
        nxt = None
        if c + 1 < _DNCH:
            idx_n, rows_n, sem_n = bufs[(c + 1) % 2]
            pltpu.sync_copy(tbl_hbm.at[pl.ds(base + (c + 1) * _DCH, _DCH)],
                            idx_n)
            nxt = pltpu.async_copy(h_hbm.at[idx_n], rows_n, sem_n)
        cur.wait()
        pltpu.sync_copy(rows_c, buf_hbm.at[pl.ds(base + c * _DCH, _DCH)])
        cur = nxt


_dispatch_call = pl.kernel(
    _dispatch_body,
    out_type=(jax.ShapeDtypeStruct((ECAP, D), jnp.float32),
              jax.ShapeDtypeStruct((_TBL,), jnp.int32)),
    mesh=plsc.VectorSubcoreMesh(core_axis_name="c", subcore_axis_name="s"),
    compiler_params=pltpu.CompilerParams(needs_layout_passes=False),
    scratch_types=[
        pltpu.VMEM((S * K,), jnp.int32),
        pltpu.VMEM((_TBL,), jnp.int32),
        pltpu.VMEM((_DCH,), jnp.int32),
        pltpu.VMEM((_DCH,), jnp.int32),
        pltpu.VMEM((_DCH, D), jnp.float32),
        pltpu.VMEM((_DCH, D), jnp.float32),
        pltpu.SemaphoreType.DMA,
        pltpu.SemaphoreType.DMA,
    ],
)


def _make_gather(nrows, chunk):
    assert nrows % (_NW * chunk) == 0
    per_w = nrows // _NW
    nchunk = per_w // chunk

    def body(tbl_hbm, idx_hbm, out_hbm, idx_v, rows_v, sem):
        wid = lax.axis_index("s") * _SC_NC + lax.axis_index("c")
        base = wid * per_w

        def go(ci, _):
            off = base + ci * chunk
            pltpu.sync_copy(idx_hbm.at[pl.ds(off, chunk)], idx_v)
            pltpu.async_copy(tbl_hbm.at[idx_v], rows_v, sem).wait()
            pltpu.sync_copy(rows_v, out_hbm.at[pl.ds(off, chunk)])
            return 0

        lax.fori_loop(0, nchunk, go, 0)

    return pl.kernel(
        body,
        out_type=jax.ShapeDtypeStruct((nrows, D), jnp.float32),
        mesh=plsc.VectorSubcoreMesh(core_axis_name="c",
                                    subcore_axis_name="s"),
        scratch_types=[
            pltpu.VMEM((chunk,), jnp.int32),
            pltpu.VMEM((chunk, D), jnp.float32),
            pltpu.SemaphoreType.DMA,
        ],
    )


_gather_out = _make_gather(2 * S, 64)


def _pe_const(s, dm):
    pos = jnp.arange(s, dtype=jnp.float32)[:, None]
    div = jnp.exp(jnp.arange(0, dm, 2, dtype=jnp.float32)
                  * (-math.log(10000.0) / dm))
    pe = jnp.zeros((s, dm), dtype=jnp.float32)
    pe = pe.at[:, 0::2].set(jnp.sin(pos * div))
    pe = pe.at[:, 1::2].set(jnp.cos(pos * div))
    return pe


def kernel(x, a0_iw, a0_ib, a0_ow, a0_ob, n0a_g, n0a_b, r_w, r_b, e_w1,
           e_b1, e_w2, e_b2, n0b_g, n0b_b, a1_iw, a1_ib, a1_ow, a1_ob,
           n1a_g, n1a_b, l1_w, l1_b, l2_w, l2_b, n1b_g, n1b_b, c_w, c_b):
    xs = x.reshape(S, D)
    pe = _pe_const(S, D)
    row = lambda a: a.reshape(1, -1)

    h, q, k, v = _qkv(xs, pe, a0_iw, row(a0_ib))
    h1 = _attn_proj_ln(q, k, v, a0_ow, row(a0_ob), h, row(n0a_g),
                       row(n0a_b))

    ds, dg, w = _router(h1, r_w, row(r_b))
    buf, _ = _dispatch_call(ds.T.reshape(-1), h1)
    y = _moe_ffn(buf, e_w1, e_b1.reshape(E, 1, HID), e_w2,
                 e_b2.reshape(E, 1, D))
    ypair = _gather_out(y, dg.T.reshape(-1))

    h2, q, k, v = _combine_qkv(h1, ypair, w, row(n0b_g), row(n0b_b),
                               a1_iw, row(a1_ib))

    h3 = _attn_proj_ln(q, k, v, a1_ow, row(a1_ob), h2, row(n1a_g),
                       row(n1a_b))
    return _ffn_pool_cls(h3, l1_w, row(l1_b), l2_w, row(l2_b), row(n1b_g),
                         row(n1b_b), c_w, row(c_b))
